# Initial kernel scaffold; baseline (speedup 1.0000x reference)
import functools

import jax
import jax.numpy as jnp
from jax import lax
from jax.experimental import pallas as pl
from jax.experimental.pallas import tpu as pltpu
from jax.experimental.pallas import tpu_sc as plsc

NC = 2
NS = 16
NW = NC * NS
K = 80
DEG_W = 8


def _sc_mesh():
    return plsc.VectorSubcoreMesh(
        core_axis_name="c", subcore_axis_name="s", num_cores=NC, num_subcores=NS
    )


def _sc_degrees(src2, dst2, ones_h, zeros_h):
    nchunk, k = src2.shape
    n = zeros_h.shape[0]
    per_tile = nchunk // NW

    def body(src_h, dst_h, ones_hbm, zeros_hbm, out_h, acc_od, acc_id,
             src_v, dst_v, ones_v, sem):
        cid = lax.axis_index("c")
        sid = lax.axis_index("s")
        wid = sid * NC + cid
        base = wid * per_tile
        pltpu.sync_copy(src_h.at[pl.ds(base, per_tile)], src_v)
        pltpu.sync_copy(dst_h.at[pl.ds(base, per_tile)], dst_v)
        pltpu.sync_copy(ones_hbm, ones_v)

        @pl.when(sid == 0)
        def _():
            pltpu.sync_copy(zeros_hbm, acc_od)
            pltpu.sync_copy(zeros_hbm, acc_id)

        plsc.subcore_barrier()

        def step(i, carry):
            pltpu.sync_copy(ones_v, acc_od.at[src_v.at[i]], add=True)
            pltpu.sync_copy(ones_v, acc_id.at[dst_v.at[i]], add=True)
            return carry

        lax.fori_loop(0, per_tile, step, 0)
        plsc.subcore_barrier()

        rows = n // NS
        pltpu.sync_copy(acc_od.at[pl.ds(sid * rows, rows)],
                        out_h.at[cid, 0, pl.ds(sid * rows, rows)])
        pltpu.sync_copy(acc_id.at[pl.ds(sid * rows, rows)],
                        out_h.at[cid, 1, pl.ds(sid * rows, rows)])

    run = pl.kernel(
        body,
        out_type=jax.ShapeDtypeStruct((NC, 2, n, DEG_W), jnp.float32),
        mesh=_sc_mesh(),
        scratch_types=[
            pltpu.VMEM_SHARED((n, DEG_W), jnp.float32),
            pltpu.VMEM_SHARED((n, DEG_W), jnp.float32),
            pltpu.VMEM((per_tile, k), jnp.int32),
            pltpu.VMEM((per_tile, k), jnp.int32),
            pltpu.VMEM((k, DEG_W), jnp.float32),
            pltpu.SemaphoreType.DMA,
        ],
    )
    return run(src2, dst2, ones_h, zeros_h)


def _sc_aggregate(src2, dst2, m, zeros_h):
    nchunk, k = src2.shape
    n, d = m.shape
    per_tile = nchunk // NW
    rows = n // NS

    def body(src_h, dst_h, m_h, zeros_hbm, out_h, acc,
             src_v, dst_v, rows_v, sem):
        cid = lax.axis_index("c")
        sid = lax.axis_index("s")
        wid = sid * NC + cid
        base = wid * per_tile
        pltpu.sync_copy(src_h.at[pl.ds(base, per_tile)], src_v)
        pltpu.sync_copy(dst_h.at[pl.ds(base, per_tile)], dst_v)
        pltpu.sync_copy(zeros_hbm, acc.at[pl.ds(sid * rows, rows)])
        plsc.subcore_barrier()

        def step(i, carry):
            pltpu.async_copy(m_h.at[src_v.at[i]], rows_v, sem).wait()
            pltpu.sync_copy(rows_v, acc.at[dst_v.at[i]], add=True)
            return carry

        lax.fori_loop(0, per_tile, step, 0)
        plsc.subcore_barrier()
        pltpu.sync_copy(acc.at[pl.ds(sid * rows, rows)],
                        out_h.at[cid, pl.ds(sid * rows, rows)])

    run = pl.kernel(
        body,
        out_type=jax.ShapeDtypeStruct((NC, n, d), jnp.float32),
        mesh=_sc_mesh(),
        scratch_types=[
            pltpu.VMEM_SHARED((n, d), jnp.float32),
            pltpu.VMEM((per_tile, k), jnp.int32),
            pltpu.VMEM((per_tile, k), jnp.int32),
            pltpu.VMEM((k, d), jnp.float32),
            pltpu.SemaphoreType.DMA,
        ],
    )
    return run(src2, dst2, m, zeros_h)


def _tc_embed(x, w, b, deg4):
    n, d = x.shape

    def body(x_ref, w_ref, b_ref, deg_ref, h_ref, m_ref, nrm_ref):
        od = deg_ref[:, 0:1] + deg_ref[:, 1:2]
        idg = deg_ref[:, 2:3] + deg_ref[:, 3:4]
        nout = jnp.where(od > 0, lax.rsqrt(od), 0.0)
        nin = jnp.where(idg > 0, lax.rsqrt(idg), 0.0)
        h = jnp.dot(x_ref[...], w_ref[...], preferred_element_type=jnp.float32)
        h = h + b_ref[...]
        h_ref[...] = h
        m_ref[...] = h * nout
        nrm_ref[...] = jnp.concatenate([nin, nout], axis=1)

    return pl.pallas_call(
        body,
        out_shape=[
            jax.ShapeDtypeStruct((n, d), jnp.float32),
            jax.ShapeDtypeStruct((n, d), jnp.float32),
            jax.ShapeDtypeStruct((n, 2), jnp.float32),
        ],
    )(x, w, b, deg4)


def _gcn_dense(p, h_prev, scl3, w, b, gamma, beta):
    n, d = h_prev.shape

    def body(p_ref, h_ref, s_ref, w_ref, b_ref, g_ref, be_ref,
             out_h_ref, out_m_ref):
        nin = s_ref[:, 0:1]
        nout = s_ref[:, 1:2]
        sn = s_ref[:, 2:3]
        agg = (p_ref[0] + p_ref[1]) * nin
        t = jnp.dot(agg, w_ref[...], preferred_element_type=jnp.float32)
        t = (t + b_ref[...]) * sn
        mean = jnp.mean(t, axis=0, keepdims=True)
        var = jnp.mean((t - mean) * (t - mean), axis=0, keepdims=True)
        hh = (t - mean) * lax.rsqrt(var + 1e-5) * g_ref[...] + be_ref[...]
        h_next = h_ref[...] + jnp.maximum(hh, 0.0)
        out_h_ref[...] = h_next
        out_m_ref[...] = h_next * nout

    return pl.pallas_call(
        body,
        out_shape=[
            jax.ShapeDtypeStruct((n, d), jnp.float32),
            jax.ShapeDtypeStruct((n, d), jnp.float32),
        ],
    )(p, h_prev, scl3, w, b, gamma, beta)


def _gcn_final(p, h_prev, scl3, w, b, gamma, beta, w_out):
    n, d = h_prev.shape
    c = w_out.shape[1]

    def body(p_ref, h_ref, s_ref, w_ref, b_ref, g_ref, be_ref, wo_ref,
             out_ref):
        nin = s_ref[:, 0:1]
        sn = s_ref[:, 2:3]
        agg = (p_ref[0] + p_ref[1]) * nin
        t = jnp.dot(agg, w_ref[...], preferred_element_type=jnp.float32)
        t = (t + b_ref[...]) * sn
        mean = jnp.mean(t, axis=0, keepdims=True)
        var = jnp.mean((t - mean) * (t - mean), axis=0, keepdims=True)
        hh = (t - mean) * lax.rsqrt(var + 1e-5) * g_ref[...] + be_ref[...]
        h_next = h_ref[...] + jnp.maximum(hh, 0.0)
        hg = jnp.mean(h_next, axis=0, keepdims=True)
        out_ref[...] = jnp.dot(hg, wo_ref[...],
                               preferred_element_type=jnp.float32)

    return pl.pallas_call(
        body,
        out_shape=jax.ShapeDtypeStruct((1, c), jnp.float32),
    )(p, h_prev, scl3, w, b, gamma, beta, w_out)


def kernel(nodes_feat, edge_index, edges_feat, nodes_num_norm_sqrt,
           edges_num_norm_sqrt, W_emb, b_emb, W1, b1, gamma1, beta1,
           W2, b2, gamma2, beta2, W_out):
    n, d = nodes_feat.shape
    e = edge_index.shape[1]
    assert e % (NW * K) == 0 and n % NS == 0

    src2 = edge_index[0].reshape(e // K, K)
    dst2 = edge_index[1].reshape(e // K, K)
    ones_h = jnp.ones((K, DEG_W), jnp.float32)
    zeros_deg = jnp.zeros((n, DEG_W), jnp.float32)
    zeros_agg = jnp.zeros((n // NS, d), jnp.float32)

    degp = _sc_degrees(src2, dst2, ones_h, zeros_deg)
    deg4 = jnp.concatenate(
        [degp[0, 0, :, 0:1], degp[1, 0, :, 0:1],
         degp[0, 1, :, 0:1], degp[1, 1, :, 0:1]], axis=1)

    h0, m1, nrm2 = _tc_embed(nodes_feat, W_emb, b_emb.reshape(1, d), deg4)
    scl3 = jnp.concatenate([nrm2, nodes_num_norm_sqrt], axis=1)

    p1 = _sc_aggregate(src2, dst2, m1, zeros_agg)
    h1, m2 = _gcn_dense(p1, h0, scl3, W1, b1.reshape(1, d),
                        gamma1.reshape(1, d), beta1.reshape(1, d))

    p2 = _sc_aggregate(src2, dst2, m2, zeros_agg)
    logits = _gcn_final(p2, h1, scl3, W2, b2.reshape(1, d),
                        gamma2.reshape(1, d), beta2.reshape(1, d), W_out)
    return logits


# speedup vs baseline: 4.4529x; 4.4529x over previous
import jax
import jax.numpy as jnp
from jax import lax
from jax.experimental import pallas as pl
from jax.experimental.pallas import tpu as pltpu
from jax.experimental.pallas import tpu_sc as plsc

NC = 2
NS = 16
NW = NC * NS
K = 80
DEG_W = 16


def _sc_mesh():
    return plsc.VectorSubcoreMesh(
        core_axis_name="c", subcore_axis_name="s", num_cores=NC, num_subcores=NS
    )


def _sc_degrees(src, dst, n_pad):
    e = src.shape[0]
    per_tile = e // NW
    steps = per_tile // K
    rows = n_pad // NS

    def body(src_h, dst_h, out_h, acc_od, acc_id, src_v, dst_v, ones_v,
             zbuf, sem):
        cid = lax.axis_index("c")
        sid = lax.axis_index("s")
        wid = sid * NC + cid
        base = wid * per_tile

        def fill_ones(i, carry):
            ones_v[pl.ds(pl.multiple_of(i * 16, 16), 16)] = jnp.ones(
                (16,), jnp.float32)
            return carry

        lax.fori_loop(0, K // 16, fill_ones, 0)

        def fill_zeros(i, carry):
            zbuf[pl.ds(pl.multiple_of(i * 16, 16), 16)] = jnp.zeros(
                (16,), jnp.float32)
            return carry

        lax.fori_loop(0, rows // 16, fill_zeros, 0)
        my0 = pl.multiple_of(sid * rows, 8)
        pltpu.sync_copy(zbuf, acc_od.at[pl.ds(my0, rows)])
        pltpu.sync_copy(zbuf, acc_id.at[pl.ds(my0, rows)])
        plsc.subcore_barrier()

        def step(i, carry):
            pltpu.sync_copy(src_h.at[pl.ds(base + i * K, K)], src_v)
            pltpu.sync_copy(dst_h.at[pl.ds(base + i * K, K)], dst_v)
            pltpu.sync_copy(ones_v, acc_od.at[src_v], add=True)
            pltpu.sync_copy(ones_v, acc_id.at[dst_v], add=True)
            return carry

        lax.fori_loop(0, steps, step, 0)
        plsc.subcore_barrier()

        my0 = pl.multiple_of(sid * rows, 8)
        o_od = pl.multiple_of(cid * 2 * n_pad + sid * rows, 8)
        o_id = pl.multiple_of(cid * 2 * n_pad + n_pad + sid * rows, 8)
        pltpu.sync_copy(acc_od.at[pl.ds(my0, rows)], zbuf)
        pltpu.sync_copy(zbuf, out_h.at[pl.ds(o_od, rows)])
        pltpu.sync_copy(acc_id.at[pl.ds(my0, rows)], zbuf)
        pltpu.sync_copy(zbuf, out_h.at[pl.ds(o_id, rows)])

    run = pl.kernel(
        body,
        out_type=jax.ShapeDtypeStruct((NC * 2 * n_pad,), jnp.float32),
        mesh=_sc_mesh(),
        scratch_types=[
            pltpu.VMEM_SHARED((n_pad,), jnp.float32),
            pltpu.VMEM_SHARED((n_pad,), jnp.float32),
            pltpu.VMEM((K,), jnp.int32),
            pltpu.VMEM((K,), jnp.int32),
            pltpu.VMEM((K,), jnp.float32),
            pltpu.VMEM((rows,), jnp.float32),
            pltpu.SemaphoreType.DMA,
        ],
    )
    return run(src, dst)


def _sc_aggregate(src, dst, m, zeros_h):
    e = src.shape[0]
    n, d = m.shape
    per_tile = e // NW
    steps = per_tile // K
    rows = zeros_h.shape[0]
    n_pad = rows * NS

    def body(src_h, dst_h, m_h, zeros_hbm, out_h, acc,
             src_v, dst_v, rows_v, sem):
        cid = lax.axis_index("c")
        sid = lax.axis_index("s")
        wid = sid * NC + cid
        base = wid * per_tile
        pltpu.sync_copy(zeros_hbm, acc.at[pl.ds(sid * rows, rows)])
        plsc.subcore_barrier()

        def step(i, carry):
            pltpu.sync_copy(src_h.at[pl.ds(base + i * K, K)], src_v)
            pltpu.sync_copy(dst_h.at[pl.ds(base + i * K, K)], dst_v)
            pltpu.async_copy(m_h.at[src_v], rows_v, sem).wait()
            pltpu.sync_copy(rows_v, acc.at[dst_v], add=True)
            return carry

        lax.fori_loop(0, steps, step, 0)
        plsc.subcore_barrier()
        pltpu.sync_copy(acc.at[pl.ds(sid * rows, rows)],
                        out_h.at[cid, pl.ds(sid * rows, rows)])

    run = pl.kernel(
        body,
        out_type=jax.ShapeDtypeStruct((NC, n_pad, d), jnp.float32),
        mesh=_sc_mesh(),
        scratch_types=[
            pltpu.VMEM_SHARED((n_pad, d), jnp.float32),
            pltpu.VMEM((K,), jnp.int32),
            pltpu.VMEM((K,), jnp.int32),
            pltpu.VMEM((K, d), jnp.float32),
            pltpu.SemaphoreType.DMA,
        ],
    )
    return run(src, dst, m, zeros_h)


def _tc_embed(x, w, b, deg4):
    n, d = x.shape

    def body(x_ref, w_ref, b_ref, deg_ref, h_ref, m_ref, nrm_ref):
        od = deg_ref[:, 0:1] + deg_ref[:, 1:2]
        idg = deg_ref[:, 2:3] + deg_ref[:, 3:4]
        nout = jnp.where(od > 0, lax.rsqrt(od), 0.0)
        nin = jnp.where(idg > 0, lax.rsqrt(idg), 0.0)
        h = jnp.dot(x_ref[...], w_ref[...], preferred_element_type=jnp.float32)
        h = h + b_ref[...]
        h_ref[...] = h
        m_ref[...] = h * nout
        nrm_ref[...] = jnp.concatenate([nin, nout], axis=1)

    return pl.pallas_call(
        body,
        out_shape=[
            jax.ShapeDtypeStruct((n, d), jnp.float32),
            jax.ShapeDtypeStruct((n, d), jnp.float32),
            jax.ShapeDtypeStruct((n, 2), jnp.float32),
        ],
    )(x, w, b, deg4)


def _gcn_dense(p, h_prev, scl3, w, b, gamma, beta):
    n, d = h_prev.shape

    def body(p_ref, h_ref, s_ref, w_ref, b_ref, g_ref, be_ref,
             out_h_ref, out_m_ref):
        nin = s_ref[:, 0:1]
        nout = s_ref[:, 1:2]
        sn = s_ref[:, 2:3]
        psum = p_ref[0] + p_ref[1]
        agg = psum[:n] * nin
        t = jnp.dot(agg, w_ref[...], preferred_element_type=jnp.float32)
        t = (t + b_ref[...]) * sn
        mean = jnp.mean(t, axis=0, keepdims=True)
        var = jnp.mean((t - mean) * (t - mean), axis=0, keepdims=True)
        hh = (t - mean) * lax.rsqrt(var + 1e-5) * g_ref[...] + be_ref[...]
        h_next = h_ref[...] + jnp.maximum(hh, 0.0)
        out_h_ref[...] = h_next
        out_m_ref[...] = h_next * nout

    return pl.pallas_call(
        body,
        out_shape=[
            jax.ShapeDtypeStruct((n, d), jnp.float32),
            jax.ShapeDtypeStruct((n, d), jnp.float32),
        ],
    )(p, h_prev, scl3, w, b, gamma, beta)


def _gcn_final(p, h_prev, scl3, w, b, gamma, beta, w_out):
    n, d = h_prev.shape
    c = w_out.shape[1]

    def body(p_ref, h_ref, s_ref, w_ref, b_ref, g_ref, be_ref, wo_ref,
             out_ref):
        nin = s_ref[:, 0:1]
        sn = s_ref[:, 2:3]
        psum = p_ref[0] + p_ref[1]
        agg = psum[:n] * nin
        t = jnp.dot(agg, w_ref[...], preferred_element_type=jnp.float32)
        t = (t + b_ref[...]) * sn
        mean = jnp.mean(t, axis=0, keepdims=True)
        var = jnp.mean((t - mean) * (t - mean), axis=0, keepdims=True)
        hh = (t - mean) * lax.rsqrt(var + 1e-5) * g_ref[...] + be_ref[...]
        h_next = h_ref[...] + jnp.maximum(hh, 0.0)
        hg = jnp.mean(h_next, axis=0, keepdims=True)
        out_ref[...] = jnp.dot(hg, wo_ref[...],
                               preferred_element_type=jnp.float32)

    return pl.pallas_call(
        body,
        out_shape=jax.ShapeDtypeStruct((1, c), jnp.float32),
    )(p, h_prev, scl3, w, b, gamma, beta, w_out)


def kernel(nodes_feat, edge_index, edges_feat, nodes_num_norm_sqrt,
           edges_num_norm_sqrt, W_emb, b_emb, W1, b1, gamma1, beta1,
           W2, b2, gamma2, beta2, W_out):
    n, d = nodes_feat.shape
    e = edge_index.shape[1]
    assert e % (NW * K) == 0

    rows = -(-n // (NS * 8)) * 8

    src = edge_index[0]
    dst = edge_index[1]
    zeros_agg = jnp.zeros((rows, d), jnp.float32)
    n_pad = rows * NS

    degp = _sc_degrees(src, dst, n_pad).reshape(NC, 2, n_pad)
    deg4 = jnp.stack(
        [degp[0, 0, :n], degp[1, 0, :n],
         degp[0, 1, :n], degp[1, 1, :n]], axis=1)

    h0, m1, nrm2 = _tc_embed(nodes_feat, W_emb, b_emb.reshape(1, d), deg4)
    scl3 = jnp.concatenate([nrm2, nodes_num_norm_sqrt], axis=1)

    p1 = _sc_aggregate(src, dst, m1, zeros_agg)
    h1, m2 = _gcn_dense(p1, h0, scl3, W1, b1.reshape(1, d),
                        gamma1.reshape(1, d), beta1.reshape(1, d))

    p2 = _sc_aggregate(src, dst, m2, zeros_agg)
    logits = _gcn_final(p2, h1, scl3, W2, b2.reshape(1, d),
                        gamma2.reshape(1, d), beta2.reshape(1, d), W_out)
    return logits
